# Initial kernel scaffold; baseline (speedup 1.0000x reference)
import jax
import jax.numpy as jnp
from jax.experimental import pallas as pl


def kernel(h_nodes, h_edges, edge_index, params):
    raise NotImplementedError("write your pallas kernel here")


# speedup vs baseline: 2.9464x; 2.9464x over previous
import functools

import jax
import jax.numpy as jnp
from jax import lax
from jax.experimental import pallas as pl
from jax.experimental.pallas import tpu as pltpu
from jax.experimental.pallas import tpu_sc as plsc

NC = 2
NS = 16
NW = NC * NS
CH = 80


def _mm(x, w):
    return lax.dot_general(x, w, (((1,), (1,)), ((), ())),
                           preferred_element_type=jnp.float32)


def _ln(x, g, b):
    mu = jnp.mean(x, axis=1, keepdims=True)
    xc = x - mu
    var = jnp.mean(xc * xc, axis=1, keepdims=True)
    return xc * lax.rsqrt(var + 1e-5) * g + b


def _sc_segment_sum(vals, recv3, zeros, np_):
    e, d = vals.shape
    niter = e // (NW * CH)
    rps = np_ // NS

    def body(vals_hbm, recv_hbm, zeros_hbm, out_hbm, idx_v, rows_v, acc):
        c = lax.axis_index("c")
        s = lax.axis_index("s")
        wid = c * NS + s
        pltpu.sync_copy(zeros_hbm, acc.at[pl.ds(s * rps, rps)])
        pltpu.sync_copy(recv_hbm.at[wid], idx_v)
        plsc.subcore_barrier()

        def step(j, carry):
            off = wid * (niter * CH) + j * CH
            pltpu.sync_copy(vals_hbm.at[pl.ds(off, CH)], rows_v)
            pltpu.sync_copy(rows_v, acc.at[idx_v.at[j]], add=True)
            return carry

        lax.fori_loop(0, niter, step, 0)
        plsc.subcore_barrier()
        pltpu.sync_copy(acc.at[pl.ds(s * rps, rps)],
                        out_hbm.at[c, pl.ds(s * rps, rps)])

    f = pl.kernel(
        body,
        out_type=jax.ShapeDtypeStruct((NC, np_, d), jnp.float32),
        mesh=plsc.VectorSubcoreMesh(core_axis_name="c", subcore_axis_name="s"),
        scratch_types=[
            pltpu.VMEM((niter, CH), jnp.int32),
            pltpu.VMEM((CH, d), jnp.float32),
            pltpu.VMEM_SHARED((np_, d), jnp.float32),
        ],
    )
    return f(vals, recv3, zeros)


def _sc_gather(tbl, snd3, e):
    n, dt = tbl.shape
    niter = e // (NW * CH)

    def body(tbl_hbm, snd_hbm, out_hbm, idx_v, rows_v):
        c = lax.axis_index("c")
        s = lax.axis_index("s")
        wid = c * NS + s
        pltpu.sync_copy(snd_hbm.at[wid], idx_v)

        def step(j, carry):
            off = wid * (niter * CH) + j * CH
            pltpu.sync_copy(tbl_hbm.at[idx_v.at[j]], rows_v)
            pltpu.sync_copy(rows_v, out_hbm.at[pl.ds(off, CH)])
            return carry

        lax.fori_loop(0, niter, step, 0)

    f = pl.kernel(
        body,
        out_type=jax.ShapeDtypeStruct((e, dt), jnp.float32),
        mesh=plsc.VectorSubcoreMesh(core_axis_name="c", subcore_axis_name="s"),
        scratch_types=[
            pltpu.VMEM((niter, CH), jnp.int32),
            pltpu.VMEM((CH, dt), jnp.float32),
        ],
    )
    return f(tbl, snd3)


def _tc_tables(h_nodes, e0, e1, w1ma, w1eb):
    n, d = h_nodes.shape
    bn = 1000

    def body(hn, p0, p1, wa, wb, out):
        out[:, :d] = _mm(hn[...], wa[...])
        out[:, d:] = _mm(p0[...] + p1[...], wb[...])

    row = pl.BlockSpec((bn, d), lambda i: (i, 0))
    wsp = pl.BlockSpec((d, d), lambda i: (0, 0))
    return pl.pallas_call(
        body,
        grid=(n // bn,),
        in_specs=[row, row, row, wsp, wsp],
        out_specs=pl.BlockSpec((bn, 2 * d), lambda i: (i, 0)),
        out_shape=jax.ShapeDtypeStruct((n, 2 * d), jnp.float32),
    )(h_nodes, e0, e1, w1ma, w1eb)


def _tc_edge_mlps(g, h_edges, wm):
    e, d = h_edges.shape
    be = 2000

    def body(g_ref, he_ref,
             w1mb, b1m, w2m, b2m, w3m, b3m, gm, bm,
             w1ed, b1e, w2e, b2e, w3e, b3e, ge, be_,
             msg_ref, eo_ref):
        he = he_ref[...]
        x = jnp.maximum(g_ref[:, :d] + _mm(he, w1mb[...]) + b1m[...], 0.)
        x = jnp.maximum(_mm(x, w2m[...]) + b2m[...], 0.)
        x = _mm(x, w3m[...]) + b3m[...]
        msg_ref[...] = _ln(x, gm[...], bm[...])
        y = jnp.maximum(g_ref[:, d:] + _mm(he, w1ed[...]) + b1e[...], 0.)
        y = jnp.maximum(_mm(y, w2e[...]) + b2e[...], 0.)
        y = _mm(y, w3e[...]) + b3e[...]
        eo_ref[...] = he + _ln(y, ge[...], be_[...])

    row = pl.BlockSpec((be, d), lambda i: (i, 0))
    wsp = pl.BlockSpec((d, d), lambda i: (0, 0))
    vsp = pl.BlockSpec((1, d), lambda i: (0, 0))
    specs = [pl.BlockSpec((be, 2 * d), lambda i: (i, 0)), row]
    for _ in range(2):
        specs += [wsp, vsp, wsp, vsp, wsp, vsp, vsp, vsp]
    return pl.pallas_call(
        body,
        grid=(e // be,),
        in_specs=specs,
        out_specs=[row, row],
        out_shape=[jax.ShapeDtypeStruct((e, d), jnp.float32),
                   jax.ShapeDtypeStruct((e, d), jnp.float32)],
    )(g, h_edges, *wm)


def _tc_node_mlp(h_nodes, q0, q1, wn):
    n, d = h_nodes.shape
    bn = 1000

    def body(hn_ref, q0_ref, q1_ref,
             w1na, w1nb, b1, w2, b2, w3, b3, gg, bb, out_ref):
        hn = hn_ref[...]
        q = q0_ref[...] + q1_ref[...]
        x = jnp.maximum(_mm(hn, w1na[...]) + _mm(q, w1nb[...]) + b1[...], 0.)
        x = jnp.maximum(_mm(x, w2[...]) + b2[...], 0.)
        x = _mm(x, w3[...]) + b3[...]
        out_ref[...] = hn + _ln(x, gg[...], bb[...])

    row = pl.BlockSpec((bn, d), lambda i: (i, 0))
    wsp = pl.BlockSpec((d, d), lambda i: (0, 0))
    vsp = pl.BlockSpec((1, d), lambda i: (0, 0))
    return pl.pallas_call(
        body,
        grid=(n // bn,),
        in_specs=[row, row, row, wsp, wsp, vsp, wsp, vsp, wsp, vsp, vsp, vsp],
        out_specs=row,
        out_shape=jax.ShapeDtypeStruct((n, d), jnp.float32),
    )(h_nodes, q0, q1, *wn)


def kernel(h_nodes, h_edges, edge_index, params):
    n, d = h_nodes.shape
    e = h_edges.shape[0]
    niter = e // (NW * CH)
    snd3 = edge_index[0].reshape(NW, niter, CH)
    rcv3 = edge_index[1].reshape(NW, niter, CH)
    np_ = -(-n // (NS * 8)) * NS * 8
    zeros = jnp.zeros((np_ // NS, d), jnp.float32)

    pm, pn, pe = params["message"], params["node"], params["edge"]
    r2 = lambda v: v.reshape(1, d)

    agg_e = _sc_segment_sum(h_edges, rcv3, zeros, np_)
    tbl = _tc_tables(h_nodes, agg_e[0, :n], agg_e[1, :n],
                     pm["W1"][:, :d], pe["W1"][:, d:])
    g = _sc_gather(tbl, snd3, e)
    wm = (pm["W1"][:, d:], r2(pm["b1"]), pm["W2"], r2(pm["b2"]),
          pm["W3"], r2(pm["b3"]), r2(pm["ln_g"]), r2(pm["ln_b"]),
          pe["W1"][:, :d] - pe["W1"][:, d:], r2(pe["b1"]),
          pe["W2"], r2(pe["b2"]), pe["W3"], r2(pe["b3"]),
          r2(pe["ln_g"]), r2(pe["ln_b"]))
    msgs, h_edges_out = _tc_edge_mlps(g, h_edges, wm)
    agg_m = _sc_segment_sum(msgs, rcv3, zeros, np_)
    wn = (pn["W1"][:, :d], pn["W1"][:, d:], r2(pn["b1"]),
          pn["W2"], r2(pn["b2"]), pn["W3"], r2(pn["b3"]),
          r2(pn["ln_g"]), r2(pn["ln_b"]))
    h_nodes_out = _tc_node_mlp(h_nodes, agg_m[0, :n], agg_m[1, :n], wn)
    return (h_nodes_out, h_edges_out)


# speedup vs baseline: 4.0156x; 1.3629x over previous
import functools

import jax
import jax.numpy as jnp
from jax import lax
from jax.experimental import pallas as pl
from jax.experimental.pallas import tpu as pltpu
from jax.experimental.pallas import tpu_sc as plsc

NC = 2
NS = 16
NW = NC * NS
CH = 80


def _mm(x, w):
    return lax.dot_general(x, w, (((1,), (1,)), ((), ())),
                           preferred_element_type=jnp.float32)


def _ln(x, g, b):
    mu = jnp.mean(x, axis=1, keepdims=True)
    xc = x - mu
    var = jnp.mean(xc * xc, axis=1, keepdims=True)
    return xc * lax.rsqrt(var + 1e-5) * g + b


NBUF = 4
NBUF_S = 3


def _sc_segment_sum(vals, recv3, zeros, np_):
    e, d = vals.shape
    niter = e // (NW * CH)
    rps = np_ // NS

    def body(vals_hbm, recv_hbm, zeros_hbm, out_hbm, idx_v, *rest):
        bufs = rest[:NBUF_S]
        sems = rest[NBUF_S:2 * NBUF_S]
        acc = rest[2 * NBUF_S]
        c = lax.axis_index("c")
        s = lax.axis_index("s")
        wid = c * NS + s
        base = wid * (niter * CH)
        pltpu.sync_copy(zeros_hbm, acc.at[pl.ds(s * rps, rps)])
        pltpu.sync_copy(recv_hbm.at[wid], idx_v)
        plsc.subcore_barrier()

        def load(j, b):
            return pltpu.make_async_copy(
                vals_hbm.at[pl.ds(base + j * CH, CH)], bufs[b], sems[b])

        for b in range(NBUF_S - 1):
            load(b, b).start()

        def step(j, carry):
            pre = j + NBUF_S - 1
            for b in range(NBUF_S):
                @pl.when(jnp.logical_and(pre < niter, pre % NBUF_S == b))
                def _(b=b):
                    load(pre, b).start()
            for b in range(NBUF_S):
                @pl.when(j % NBUF_S == b)
                def _(b=b):
                    load(j, b).wait()
                    pltpu.sync_copy(bufs[b], acc.at[idx_v.at[j]], add=True)
            return carry

        lax.fori_loop(0, niter, step, 0)
        plsc.subcore_barrier()
        pltpu.sync_copy(acc.at[pl.ds(s * rps, rps)],
                        out_hbm.at[c, pl.ds(s * rps, rps)])

    f = pl.kernel(
        body,
        out_type=jax.ShapeDtypeStruct((NC, np_, d), jnp.float32),
        mesh=plsc.VectorSubcoreMesh(core_axis_name="c", subcore_axis_name="s"),
        scratch_types=(
            [pltpu.VMEM((niter, CH), jnp.int32)]
            + [pltpu.VMEM((CH, d), jnp.float32)] * NBUF_S
            + [pltpu.SemaphoreType.DMA] * NBUF_S
            + [pltpu.VMEM_SHARED((np_, d), jnp.float32)]
        ),
    )
    return f(vals, recv3, zeros)


def _sc_gather(tbl, snd3, e):
    n, dt = tbl.shape
    niter = e // (NW * CH)

    def body(tbl_hbm, snd_hbm, out_hbm, idx_v, *rest):
        bufs = rest[:NBUF]
        gsems = rest[NBUF:2 * NBUF]
        wsems = rest[2 * NBUF:3 * NBUF]
        c = lax.axis_index("c")
        s = lax.axis_index("s")
        wid = c * NS + s
        base = wid * (niter * CH)
        pltpu.sync_copy(snd_hbm.at[wid], idx_v)

        def rd(j, b):
            return pltpu.make_async_copy(tbl_hbm.at[idx_v.at[j]],
                                         bufs[b], gsems[b])

        def wr(j, b):
            return pltpu.make_async_copy(
                bufs[b], out_hbm.at[pl.ds(base + j * CH, CH)], wsems[b])

        for b in range(NBUF - 1):
            rd(b, b).start()

        def step(j, carry):
            pre = j + NBUF - 1
            for b in range(NBUF):
                @pl.when(jnp.logical_and(pre < niter, pre % NBUF == b))
                def _(b=b):
                    @pl.when(pre >= NBUF)
                    def _():
                        wr(pre - NBUF, b).wait()
                    rd(pre, b).start()
            for b in range(NBUF):
                @pl.when(j % NBUF == b)
                def _(b=b):
                    rd(j, b).wait()
                    wr(j, b).start()
            return carry

        lax.fori_loop(0, niter, step, 0)
        for j in range(max(0, niter - NBUF), niter):
            wr(j, j % NBUF).wait()

    f = pl.kernel(
        body,
        out_type=jax.ShapeDtypeStruct((e, dt), jnp.float32),
        mesh=plsc.VectorSubcoreMesh(core_axis_name="c", subcore_axis_name="s"),
        scratch_types=(
            [pltpu.VMEM((niter, CH), jnp.int32)]
            + [pltpu.VMEM((CH, dt), jnp.float32)] * NBUF
            + [pltpu.SemaphoreType.DMA] * (2 * NBUF)
        ),
    )
    return f(tbl, snd3)


def _tc_tables(h_nodes, e0, e1, w1ma, w1eb):
    n, d = h_nodes.shape
    bn = 1000

    def body(hn, p0, p1, wa, wb, out):
        out[:, :d] = _mm(hn[...], wa[...])
        out[:, d:] = _mm(p0[...] + p1[...], wb[...])

    row = pl.BlockSpec((bn, d), lambda i: (i, 0))
    wsp = pl.BlockSpec((d, d), lambda i: (0, 0))
    return pl.pallas_call(
        body,
        grid=(n // bn,),
        in_specs=[row, row, row, wsp, wsp],
        out_specs=pl.BlockSpec((bn, 2 * d), lambda i: (i, 0)),
        out_shape=jax.ShapeDtypeStruct((n, 2 * d), jnp.float32),
    )(h_nodes, e0, e1, w1ma, w1eb)


def _tc_edge_mlps(g, h_edges, wm):
    e, d = h_edges.shape
    be = 2000

    def body(g_ref, he_ref,
             w1mb, b1m, w2m, b2m, w3m, b3m, gm, bm,
             w1ed, b1e, w2e, b2e, w3e, b3e, ge, be_,
             msg_ref, eo_ref):
        he = he_ref[...]
        x = jnp.maximum(g_ref[:, :d] + _mm(he, w1mb[...]) + b1m[...], 0.)
        x = jnp.maximum(_mm(x, w2m[...]) + b2m[...], 0.)
        x = _mm(x, w3m[...]) + b3m[...]
        msg_ref[...] = _ln(x, gm[...], bm[...])
        y = jnp.maximum(g_ref[:, d:] + _mm(he, w1ed[...]) + b1e[...], 0.)
        y = jnp.maximum(_mm(y, w2e[...]) + b2e[...], 0.)
        y = _mm(y, w3e[...]) + b3e[...]
        eo_ref[...] = he + _ln(y, ge[...], be_[...])

    row = pl.BlockSpec((be, d), lambda i: (i, 0))
    wsp = pl.BlockSpec((d, d), lambda i: (0, 0))
    vsp = pl.BlockSpec((1, d), lambda i: (0, 0))
    specs = [pl.BlockSpec((be, 2 * d), lambda i: (i, 0)), row]
    for _ in range(2):
        specs += [wsp, vsp, wsp, vsp, wsp, vsp, vsp, vsp]
    return pl.pallas_call(
        body,
        grid=(e // be,),
        in_specs=specs,
        out_specs=[row, row],
        out_shape=[jax.ShapeDtypeStruct((e, d), jnp.float32),
                   jax.ShapeDtypeStruct((e, d), jnp.float32)],
    )(g, h_edges, *wm)


def _tc_node_mlp(h_nodes, q0, q1, wn):
    n, d = h_nodes.shape
    bn = 1000

    def body(hn_ref, q0_ref, q1_ref,
             w1na, w1nb, b1, w2, b2, w3, b3, gg, bb, out_ref):
        hn = hn_ref[...]
        q = q0_ref[...] + q1_ref[...]
        x = jnp.maximum(_mm(hn, w1na[...]) + _mm(q, w1nb[...]) + b1[...], 0.)
        x = jnp.maximum(_mm(x, w2[...]) + b2[...], 0.)
        x = _mm(x, w3[...]) + b3[...]
        out_ref[...] = hn + _ln(x, gg[...], bb[...])

    row = pl.BlockSpec((bn, d), lambda i: (i, 0))
    wsp = pl.BlockSpec((d, d), lambda i: (0, 0))
    vsp = pl.BlockSpec((1, d), lambda i: (0, 0))
    return pl.pallas_call(
        body,
        grid=(n // bn,),
        in_specs=[row, row, row, wsp, wsp, vsp, wsp, vsp, wsp, vsp, vsp, vsp],
        out_specs=row,
        out_shape=jax.ShapeDtypeStruct((n, d), jnp.float32),
    )(h_nodes, q0, q1, *wn)


def kernel(h_nodes, h_edges, edge_index, params):
    n, d = h_nodes.shape
    e = h_edges.shape[0]
    niter = e // (NW * CH)
    snd3 = edge_index[0].reshape(NW, niter, CH)
    rcv3 = edge_index[1].reshape(NW, niter, CH)
    np_ = -(-n // (NS * 8)) * NS * 8
    zeros = jnp.zeros((np_ // NS, d), jnp.float32)

    pm, pn, pe = params["message"], params["node"], params["edge"]
    r2 = lambda v: v.reshape(1, d)

    agg_e = _sc_segment_sum(h_edges, rcv3, zeros, np_)
    tbl = _tc_tables(h_nodes, agg_e[0, :n], agg_e[1, :n],
                     pm["W1"][:, :d], pe["W1"][:, d:])
    g = _sc_gather(tbl, snd3, e)
    wm = (pm["W1"][:, d:], r2(pm["b1"]), pm["W2"], r2(pm["b2"]),
          pm["W3"], r2(pm["b3"]), r2(pm["ln_g"]), r2(pm["ln_b"]),
          pe["W1"][:, :d] - pe["W1"][:, d:], r2(pe["b1"]),
          pe["W2"], r2(pe["b2"]), pe["W3"], r2(pe["b3"]),
          r2(pe["ln_g"]), r2(pe["ln_b"]))
    msgs, h_edges_out = _tc_edge_mlps(g, h_edges, wm)
    agg_m = _sc_segment_sum(msgs, rcv3, zeros, np_)
    wn = (pn["W1"][:, :d], pn["W1"][:, d:], r2(pn["b1"]),
          pn["W2"], r2(pn["b2"]), pn["W3"], r2(pn["b3"]),
          r2(pn["ln_g"]), r2(pn["ln_b"]))
    h_nodes_out = _tc_node_mlp(h_nodes, agg_m[0, :n], agg_m[1, :n], wn)
    return (h_nodes_out, h_edges_out)


# speedup vs baseline: 4.6651x; 1.1617x over previous
import functools

import jax
import jax.numpy as jnp
from jax import lax
from jax.experimental import pallas as pl
from jax.experimental.pallas import tpu as pltpu
from jax.experimental.pallas import tpu_sc as plsc

NC = 2
NS = 16
NW = NC * NS
CH = 80


def _mm(x, w):
    return lax.dot_general(x.astype(jnp.bfloat16), w.astype(jnp.bfloat16),
                           (((1,), (1,)), ((), ())),
                           preferred_element_type=jnp.float32)


def _ln(x, g, b):
    mu = jnp.mean(x, axis=1, keepdims=True)
    xc = x - mu
    var = jnp.mean(xc * xc, axis=1, keepdims=True)
    return xc * lax.rsqrt(var + 1e-5) * g + b


NBUF = 4
NBUF_S = 3


def _sc_segment_sum(vals, recv3, zeros, np_):
    e, d = vals.shape
    niter = e // (NW * CH)
    rps = np_ // NS

    def body(vals_hbm, recv_hbm, zeros_hbm, out_hbm, idx_v, *rest):
        bufs = rest[:NBUF_S]
        sems = rest[NBUF_S:2 * NBUF_S]
        acc = rest[2 * NBUF_S]
        c = lax.axis_index("c")
        s = lax.axis_index("s")
        wid = c * NS + s
        base = wid * (niter * CH)
        pltpu.sync_copy(zeros_hbm, acc.at[pl.ds(s * rps, rps)])
        pltpu.sync_copy(recv_hbm.at[wid], idx_v)
        plsc.subcore_barrier()

        def load(j, b):
            return pltpu.make_async_copy(
                vals_hbm.at[pl.ds(base + j * CH, CH)], bufs[b], sems[b])

        for b in range(NBUF_S - 1):
            load(b, b).start()

        def step(j, carry):
            pre = j + NBUF_S - 1
            for b in range(NBUF_S):
                @pl.when(jnp.logical_and(pre < niter, pre % NBUF_S == b))
                def _(b=b):
                    load(pre, b).start()
            for b in range(NBUF_S):
                @pl.when(j % NBUF_S == b)
                def _(b=b):
                    load(j, b).wait()
                    pltpu.sync_copy(bufs[b], acc.at[idx_v.at[j]], add=True)
            return carry

        lax.fori_loop(0, niter, step, 0)
        plsc.subcore_barrier()
        pltpu.sync_copy(acc.at[pl.ds(s * rps, rps)],
                        out_hbm.at[c, pl.ds(s * rps, rps)])

    f = pl.kernel(
        body,
        out_type=jax.ShapeDtypeStruct((NC, np_, d), jnp.float32),
        mesh=plsc.VectorSubcoreMesh(core_axis_name="c", subcore_axis_name="s"),
        scratch_types=(
            [pltpu.VMEM((niter, CH), jnp.int32)]
            + [pltpu.VMEM((CH, d), jnp.float32)] * NBUF_S
            + [pltpu.SemaphoreType.DMA] * NBUF_S
            + [pltpu.VMEM_SHARED((np_, d), jnp.float32)]
        ),
    )
    return f(vals, recv3, zeros)


def _sc_gather(tbl, snd3, e):
    n, dt = tbl.shape
    niter = e // (NW * CH)

    def body(tbl_hbm, snd_hbm, out_hbm, idx_v, *rest):
        bufs = rest[:NBUF]
        gsems = rest[NBUF:2 * NBUF]
        wsems = rest[2 * NBUF:3 * NBUF]
        c = lax.axis_index("c")
        s = lax.axis_index("s")
        wid = c * NS + s
        base = wid * (niter * CH)
        pltpu.sync_copy(snd_hbm.at[wid], idx_v)

        def rd(j, b):
            return pltpu.make_async_copy(tbl_hbm.at[idx_v.at[j]],
                                         bufs[b], gsems[b])

        def wr(j, b):
            return pltpu.make_async_copy(
                bufs[b], out_hbm.at[pl.ds(base + j * CH, CH)], wsems[b])

        for b in range(NBUF - 1):
            rd(b, b).start()

        def step(j, carry):
            pre = j + NBUF - 1
            for b in range(NBUF):
                @pl.when(jnp.logical_and(pre < niter, pre % NBUF == b))
                def _(b=b):
                    @pl.when(pre >= NBUF)
                    def _():
                        wr(pre - NBUF, b).wait()
                    rd(pre, b).start()
            for b in range(NBUF):
                @pl.when(j % NBUF == b)
                def _(b=b):
                    rd(j, b).wait()
                    wr(j, b).start()
            return carry

        lax.fori_loop(0, niter, step, 0)
        for j in range(max(0, niter - NBUF), niter):
            wr(j, j % NBUF).wait()

    f = pl.kernel(
        body,
        out_type=jax.ShapeDtypeStruct((e, dt), jnp.float32),
        mesh=plsc.VectorSubcoreMesh(core_axis_name="c", subcore_axis_name="s"),
        scratch_types=(
            [pltpu.VMEM((niter, CH), jnp.int32)]
            + [pltpu.VMEM((CH, dt), jnp.float32)] * NBUF
            + [pltpu.SemaphoreType.DMA] * (2 * NBUF)
        ),
    )
    return f(tbl, snd3)


def _tc_tables(h_nodes, e0, e1, w1ma, w1eb):
    n, d = h_nodes.shape
    bn = 1000

    def body(hn, p0, p1, wa, wb, out):
        a = _mm(hn[...], wa[...])
        p = _mm(p0[...] + p1[...], wb[...])
        au = lax.bitcast_convert_type(a.astype(jnp.bfloat16),
                                      jnp.uint16).astype(jnp.uint32)
        pu = lax.bitcast_convert_type(p.astype(jnp.bfloat16),
                                      jnp.uint16).astype(jnp.uint32)
        out[...] = lax.bitcast_convert_type((au << 16) | pu, jnp.float32)

    row = pl.BlockSpec((bn, d), lambda i: (i, 0))
    wsp = pl.BlockSpec((d, d), lambda i: (0, 0))
    return pl.pallas_call(
        body,
        grid=(n // bn,),
        in_specs=[row, row, row, wsp, wsp],
        out_specs=row,
        out_shape=jax.ShapeDtypeStruct((n, d), jnp.float32),
    )(h_nodes, e0, e1, w1ma, w1eb)


def _tc_edge_mlps(g, h_edges, wm):
    e, d = h_edges.shape
    be = 2000

    def body(g_ref, he_ref,
             w1mb, b1m, w2m, b2m, w3m, b3m, gm, bm,
             w1ed, b1e, w2e, b2e, w3e, b3e, ge, be_,
             msg_ref, eo_ref):
        he = he_ref[...]
        gw = lax.bitcast_convert_type(g_ref[...], jnp.uint32)
        ga = lax.bitcast_convert_type((gw >> 16).astype(jnp.uint16),
                                      jnp.bfloat16).astype(jnp.float32)
        gp = lax.bitcast_convert_type(gw.astype(jnp.uint16),
                                      jnp.bfloat16).astype(jnp.float32)
        x = jnp.maximum(ga + _mm(he, w1mb[...]) + b1m[...], 0.)
        x = jnp.maximum(_mm(x, w2m[...]) + b2m[...], 0.)
        x = _mm(x, w3m[...]) + b3m[...]
        msg_ref[...] = _ln(x, gm[...], bm[...])
        y = jnp.maximum(gp + _mm(he, w1ed[...]) + b1e[...], 0.)
        y = jnp.maximum(_mm(y, w2e[...]) + b2e[...], 0.)
        y = _mm(y, w3e[...]) + b3e[...]
        eo_ref[...] = he + _ln(y, ge[...], be_[...])

    row = pl.BlockSpec((be, d), lambda i: (i, 0))
    wsp = pl.BlockSpec((d, d), lambda i: (0, 0))
    vsp = pl.BlockSpec((1, d), lambda i: (0, 0))
    specs = [row, row]
    for _ in range(2):
        specs += [wsp, vsp, wsp, vsp, wsp, vsp, vsp, vsp]
    return pl.pallas_call(
        body,
        grid=(e // be,),
        in_specs=specs,
        out_specs=[row, row],
        out_shape=[jax.ShapeDtypeStruct((e, d), jnp.float32),
                   jax.ShapeDtypeStruct((e, d), jnp.float32)],
    )(g, h_edges, *wm)


def _tc_node_mlp(h_nodes, q0, q1, wn):
    n, d = h_nodes.shape
    bn = 1000

    def body(hn_ref, q0_ref, q1_ref,
             w1na, w1nb, b1, w2, b2, w3, b3, gg, bb, out_ref):
        hn = hn_ref[...]
        q = q0_ref[...] + q1_ref[...]
        x = jnp.maximum(_mm(hn, w1na[...]) + _mm(q, w1nb[...]) + b1[...], 0.)
        x = jnp.maximum(_mm(x, w2[...]) + b2[...], 0.)
        x = _mm(x, w3[...]) + b3[...]
        out_ref[...] = hn + _ln(x, gg[...], bb[...])

    row = pl.BlockSpec((bn, d), lambda i: (i, 0))
    wsp = pl.BlockSpec((d, d), lambda i: (0, 0))
    vsp = pl.BlockSpec((1, d), lambda i: (0, 0))
    return pl.pallas_call(
        body,
        grid=(n // bn,),
        in_specs=[row, row, row, wsp, wsp, vsp, wsp, vsp, wsp, vsp, vsp, vsp],
        out_specs=row,
        out_shape=jax.ShapeDtypeStruct((n, d), jnp.float32),
    )(h_nodes, q0, q1, *wn)


def kernel(h_nodes, h_edges, edge_index, params):
    n, d = h_nodes.shape
    e = h_edges.shape[0]
    niter = e // (NW * CH)
    snd3 = edge_index[0].reshape(NW, niter, CH)
    rcv3 = edge_index[1].reshape(NW, niter, CH)
    np_ = -(-n // (NS * 8)) * NS * 8
    zeros = jnp.zeros((np_ // NS, d), jnp.float32)

    pm, pn, pe = params["message"], params["node"], params["edge"]
    r2 = lambda v: v.reshape(1, d)

    agg_e = _sc_segment_sum(h_edges, rcv3, zeros, np_)
    tbl = _tc_tables(h_nodes, agg_e[0, :n], agg_e[1, :n],
                     pm["W1"][:, :d], pe["W1"][:, d:])
    g = _sc_gather(tbl, snd3, e)
    wm = (pm["W1"][:, d:], r2(pm["b1"]), pm["W2"], r2(pm["b2"]),
          pm["W3"], r2(pm["b3"]), r2(pm["ln_g"]), r2(pm["ln_b"]),
          pe["W1"][:, :d] - pe["W1"][:, d:], r2(pe["b1"]),
          pe["W2"], r2(pe["b2"]), pe["W3"], r2(pe["b3"]),
          r2(pe["ln_g"]), r2(pe["ln_b"]))
    msgs, h_edges_out = _tc_edge_mlps(g, h_edges, wm)
    agg_m = _sc_segment_sum(msgs, rcv3, zeros, np_)
    wn = (pn["W1"][:, :d], pn["W1"][:, d:], r2(pn["b1"]),
          pn["W2"], r2(pn["b2"]), pn["W3"], r2(pn["b3"]),
          r2(pn["ln_g"]), r2(pn["ln_b"]))
    h_nodes_out = _tc_node_mlp(h_nodes, agg_m[0, :n], agg_m[1, :n], wn)
    return (h_nodes_out, h_edges_out)


# speedup vs baseline: 4.9129x; 1.0531x over previous
import jax
import jax.numpy as jnp
from jax import lax
from jax.experimental import pallas as pl
from jax.experimental.pallas import tpu as pltpu
from jax.experimental.pallas import tpu_sc as plsc

NC = 2
NS = 16
NW = NC * NS

NBUF = 4
NBUF_S = 3


def _mm(x, w):
    return lax.dot_general(x.astype(jnp.bfloat16), w.astype(jnp.bfloat16),
                           (((1,), (1,)), ((), ())),
                           preferred_element_type=jnp.float32)


def _ln(x, g, b):
    mu = jnp.mean(x, axis=1, keepdims=True)
    xc = x - mu
    var = jnp.mean(xc * xc, axis=1, keepdims=True)
    return xc * lax.rsqrt(var + 1e-5) * g + b


def _sc_segment_sum(vals, recv3, zeros, np_, ch):
    e, d = vals.shape
    niter = e // (NW * ch)
    rps = np_ // NS

    def body(vals_hbm, recv_hbm, zeros_hbm, out_hbm, idx_v, *rest):
        bufs = rest[:NBUF_S]
        sems = rest[NBUF_S:2 * NBUF_S]
        acc = rest[2 * NBUF_S]
        c = lax.axis_index("c")
        s = lax.axis_index("s")
        wid = c * NS + s
        base = wid * (niter * ch)
        pltpu.sync_copy(zeros_hbm, acc.at[pl.ds(s * rps, rps)])
        pltpu.sync_copy(recv_hbm.at[wid], idx_v)
        plsc.subcore_barrier()

        def load(j, b):
            return pltpu.make_async_copy(
                vals_hbm.at[pl.ds(base + j * ch, ch)], bufs[b], sems[b])

        for b in range(NBUF_S - 1):
            load(b, b).start()

        def step(j, carry):
            pre = j + NBUF_S - 1
            for b in range(NBUF_S):
                @pl.when(jnp.logical_and(pre < niter, pre % NBUF_S == b))
                def _(b=b):
                    load(pre, b).start()
            for b in range(NBUF_S):
                @pl.when(j % NBUF_S == b)
                def _(b=b):
                    load(j, b).wait()
                    pltpu.sync_copy(bufs[b], acc.at[idx_v.at[j]], add=True)
            return carry

        lax.fori_loop(0, niter, step, 0)
        plsc.subcore_barrier()
        pltpu.sync_copy(acc.at[pl.ds(s * rps, rps)],
                        out_hbm.at[c, pl.ds(s * rps, rps)])

    f = pl.kernel(
        body,
        out_type=jax.ShapeDtypeStruct((NC, np_, d), jnp.float32),
        mesh=plsc.VectorSubcoreMesh(core_axis_name="c", subcore_axis_name="s"),
        scratch_types=(
            [pltpu.VMEM((niter, ch), jnp.int32)]
            + [pltpu.VMEM((ch, d), jnp.float32)] * NBUF_S
            + [pltpu.SemaphoreType.DMA] * NBUF_S
            + [pltpu.VMEM_SHARED((np_, d), jnp.float32)]
        ),
    )
    return f(vals, recv3, zeros)


def _sc_gather(tbl, snd3, e, ch):
    n, dt = tbl.shape
    niter = e // (NW * ch)

    def body(tbl_hbm, snd_hbm, out_hbm, idx_v, *rest):
        bufs = rest[:NBUF]
        gsems = rest[NBUF:2 * NBUF]
        wsems = rest[2 * NBUF:3 * NBUF]
        c = lax.axis_index("c")
        s = lax.axis_index("s")
        wid = c * NS + s
        base = wid * (niter * ch)
        pltpu.sync_copy(snd_hbm.at[wid], idx_v)

        def rd(j, b):
            return pltpu.make_async_copy(tbl_hbm.at[idx_v.at[j]],
                                         bufs[b], gsems[b])

        def wr(j, b):
            return pltpu.make_async_copy(
                bufs[b], out_hbm.at[pl.ds(base + j * ch, ch)], wsems[b])

        for b in range(NBUF - 1):
            rd(b, b).start()

        def step(j, carry):
            pre = j + NBUF - 1
            for b in range(NBUF):
                @pl.when(jnp.logical_and(pre < niter, pre % NBUF == b))
                def _(b=b):
                    @pl.when(pre >= NBUF)
                    def _():
                        wr(pre - NBUF, b).wait()
                    rd(pre, b).start()
            for b in range(NBUF):
                @pl.when(j % NBUF == b)
                def _(b=b):
                    rd(j, b).wait()
                    wr(j, b).start()
            return carry

        lax.fori_loop(0, niter, step, 0)
        for j in range(max(0, niter - NBUF), niter):
            wr(j, j % NBUF).wait()

    f = pl.kernel(
        body,
        out_type=jax.ShapeDtypeStruct((e, dt), jnp.float32),
        mesh=plsc.VectorSubcoreMesh(core_axis_name="c", subcore_axis_name="s"),
        scratch_types=(
            [pltpu.VMEM((niter, ch), jnp.int32)]
            + [pltpu.VMEM((ch, dt), jnp.float32)] * NBUF
            + [pltpu.SemaphoreType.DMA] * (2 * NBUF)
        ),
    )
    return f(tbl, snd3)


def _tc_tables(h_nodes, e0, e1, w1ma, w1eb):
    n, d = h_nodes.shape
    bn = 1000

    def body(hn, p0, p1, wa, wb, out):
        a = _mm(hn[...], wa[...])
        p = _mm(p0[...] + p1[...], wb[...])
        au = lax.bitcast_convert_type(a.astype(jnp.bfloat16),
                                      jnp.uint16).astype(jnp.uint32)
        pu = lax.bitcast_convert_type(p.astype(jnp.bfloat16),
                                      jnp.uint16).astype(jnp.uint32)
        out[...] = lax.bitcast_convert_type((au << 16) | pu, jnp.float32)

    row = pl.BlockSpec((bn, d), lambda i: (i, 0))
    wsp = pl.BlockSpec((d, d), lambda i: (0, 0))
    return pl.pallas_call(
        body,
        grid=(n // bn,),
        in_specs=[row, row, row, wsp, wsp],
        out_specs=row,
        out_shape=jax.ShapeDtypeStruct((n, d), jnp.float32),
    )(h_nodes, e0, e1, w1ma, w1eb)


def _tc_edge_mlps(g, h_edges, wm, offb, eo_prev):
    e2, d = g.shape
    e = h_edges.shape[0]
    be = 2000

    def body(g_ref, he_ref, wcat, b1m, w2m, b2m, w3m, b3m, gm, bm,
             b1e, w2e, b2e, w3e, b3e, ge, be_, *rest):
        msg_ref, eo_ref = rest[-2], rest[-1]
        he = he_ref[...]
        hw = _mm(he, wcat[...])
        gw = lax.bitcast_convert_type(g_ref[...], jnp.uint32)
        ga = lax.bitcast_convert_type((gw >> 16).astype(jnp.uint16),
                                      jnp.bfloat16).astype(jnp.float32)
        gp = lax.bitcast_convert_type(gw.astype(jnp.uint16),
                                      jnp.bfloat16).astype(jnp.float32)
        x = jnp.maximum(ga + hw[:, :d] + b1m[...], 0.)
        x = jnp.maximum(_mm(x, w2m[...]) + b2m[...], 0.)
        x = _mm(x, w3m[...]) + b3m[...]
        msg_ref[...] = _ln(x, gm[...], bm[...])
        y = jnp.maximum(gp + hw[:, d:] + b1e[...], 0.)
        y = jnp.maximum(_mm(y, w2e[...]) + b2e[...], 0.)
        y = _mm(y, w3e[...]) + b3e[...]
        eo_ref[...] = he + _ln(y, ge[...], be_[...])

    row = pl.BlockSpec((be, d), lambda i: (i, 0))
    rowo = pl.BlockSpec((be, d), lambda i: (i + offb, 0))
    wsp2 = pl.BlockSpec((2 * d, d), lambda i: (0, 0))
    wsp = pl.BlockSpec((d, d), lambda i: (0, 0))
    vsp = pl.BlockSpec((1, d), lambda i: (0, 0))
    specs = [row, rowo, wsp2,
             vsp, wsp, vsp, wsp, vsp, vsp, vsp,
             vsp, wsp, vsp, wsp, vsp, vsp, vsp]
    args = [g, h_edges] + list(wm)
    kwargs = {}
    if eo_prev is not None:
        specs.append(pl.BlockSpec(memory_space=pl.ANY))
        args.append(eo_prev)
        kwargs["input_output_aliases"] = {17: 1}
    return pl.pallas_call(
        body,
        grid=(e2 // be,),
        in_specs=specs,
        out_specs=[row, rowo],
        out_shape=[jax.ShapeDtypeStruct((e2, d), jnp.float32),
                   jax.ShapeDtypeStruct((e, d), jnp.float32)],
        **kwargs,
    )(*args)


def _tc_node_mlp(h_nodes, qs, wn):
    n, d = h_nodes.shape
    bn = 1000

    def body(hn_ref, q0_ref, q1_ref, q2_ref, q3_ref,
             w1na, w1nb, b1, w2, b2, w3, b3, gg, bb, out_ref):
        hn = hn_ref[...]
        q = (q0_ref[...] + q1_ref[...]) + (q2_ref[...] + q3_ref[...])
        x = jnp.maximum(_mm(hn, w1na[...]) + _mm(q, w1nb[...]) + b1[...], 0.)
        x = jnp.maximum(_mm(x, w2[...]) + b2[...], 0.)
        x = _mm(x, w3[...]) + b3[...]
        out_ref[...] = hn + _ln(x, gg[...], bb[...])

    row = pl.BlockSpec((bn, d), lambda i: (i, 0))
    wsp = pl.BlockSpec((d, d), lambda i: (0, 0))
    vsp = pl.BlockSpec((1, d), lambda i: (0, 0))
    return pl.pallas_call(
        body,
        grid=(n // bn,),
        in_specs=[row, row, row, row, row,
                  wsp, wsp, vsp, wsp, vsp, wsp, vsp, vsp, vsp],
        out_specs=row,
        out_shape=jax.ShapeDtypeStruct((n, d), jnp.float32),
    )(h_nodes, *qs, *wn)


def kernel(h_nodes, h_edges, edge_index, params):
    n, d = h_nodes.shape
    e = h_edges.shape[0]
    e2 = e // 2
    ch_full, ch_half = 80, 40
    snd, rcv = edge_index[0], edge_index[1]
    rcv3 = rcv.reshape(NW, e // (NW * ch_full), ch_full)
    snd3a = snd[:e2].reshape(NW, e2 // (NW * ch_half), ch_half)
    snd3b = snd[e2:].reshape(NW, e2 // (NW * ch_half), ch_half)
    rcv3a = rcv[:e2].reshape(NW, e2 // (NW * ch_half), ch_half)
    rcv3b = rcv[e2:].reshape(NW, e2 // (NW * ch_half), ch_half)
    np_ = -(-n // (NS * 8)) * NS * 8
    zeros = jnp.zeros((np_ // NS, d), jnp.float32)

    pm, pn, pe = params["message"], params["node"], params["edge"]
    r2 = lambda v: v.reshape(1, d)

    agg_e = _sc_segment_sum(h_edges, rcv3, zeros, np_, ch_full)
    tbl = _tc_tables(h_nodes, agg_e[0, :n], agg_e[1, :n],
                     pm["W1"][:, :d], pe["W1"][:, d:])
    g1 = _sc_gather(tbl, snd3a, e2, ch_half)
    g2 = _sc_gather(tbl, snd3b, e2, ch_half)
    wcat = jnp.concatenate([pm["W1"][:, d:],
                            pe["W1"][:, :d] - pe["W1"][:, d:]], axis=0)
    wm = (wcat, r2(pm["b1"]), pm["W2"], r2(pm["b2"]),
          pm["W3"], r2(pm["b3"]), r2(pm["ln_g"]), r2(pm["ln_b"]),
          r2(pe["b1"]), pe["W2"], r2(pe["b2"]), pe["W3"], r2(pe["b3"]),
          r2(pe["ln_g"]), r2(pe["ln_b"]))
    msgs1, eo1 = _tc_edge_mlps(g1, h_edges, wm, 0, None)
    msgs2, h_edges_out = _tc_edge_mlps(g2, h_edges, wm, e2 // 2000, eo1)
    am1 = _sc_segment_sum(msgs1, rcv3a, zeros, np_, ch_half)
    am2 = _sc_segment_sum(msgs2, rcv3b, zeros, np_, ch_half)
    wn = (pn["W1"][:, :d], pn["W1"][:, d:], r2(pn["b1"]),
          pn["W2"], r2(pn["b2"]), pn["W3"], r2(pn["b3"]),
          r2(pn["ln_g"]), r2(pn["ln_b"]))
    qs = [am1[0, :n], am1[1, :n], am2[0, :n], am2[1, :n]]
    h_nodes_out = _tc_node_mlp(h_nodes, qs, wn)
    return (h_nodes_out, h_edges_out)


# speedup vs baseline: 5.0495x; 1.0278x over previous
import jax
import jax.numpy as jnp
from jax import lax
from jax.experimental import pallas as pl
from jax.experimental.pallas import tpu as pltpu
from jax.experimental.pallas import tpu_sc as plsc

NC = 2
NS = 16
NW = NC * NS

NBUF = 4
NBUF_S = 3


def _mm(x, w):
    return lax.dot_general(x.astype(jnp.bfloat16), w.astype(jnp.bfloat16),
                           (((1,), (1,)), ((), ())),
                           preferred_element_type=jnp.float32)


def _ln(x, g, b):
    mu = jnp.mean(x, axis=1, keepdims=True)
    xc = x - mu
    var = jnp.mean(xc * xc, axis=1, keepdims=True)
    return xc * lax.rsqrt(var + 1e-5) * g + b


def _sc_segment_sum(vals, recv3, zeros, np_, ch):
    e, d = vals.shape
    niter = e // (NW * ch)
    rps = np_ // NS

    def body(vals_hbm, recv_hbm, zeros_hbm, out_hbm, idx_v, *rest):
        bufs = rest[:NBUF_S]
        sems = rest[NBUF_S:2 * NBUF_S]
        acc = rest[2 * NBUF_S]
        c = lax.axis_index("c")
        s = lax.axis_index("s")
        wid = c * NS + s
        base = wid * (niter * ch)
        pltpu.sync_copy(zeros_hbm, acc.at[pl.ds(s * rps, rps)])
        pltpu.sync_copy(recv_hbm.at[wid], idx_v)
        plsc.subcore_barrier()

        def load(j, b):
            return pltpu.make_async_copy(
                vals_hbm.at[pl.ds(base + j * ch, ch)], bufs[b], sems[b])

        for b in range(NBUF_S - 1):
            load(b, b).start()

        def step(j, carry):
            pre = j + NBUF_S - 1
            for b in range(NBUF_S):
                @pl.when(jnp.logical_and(pre < niter, pre % NBUF_S == b))
                def _(b=b):
                    load(pre, b).start()
            for b in range(NBUF_S):
                @pl.when(j % NBUF_S == b)
                def _(b=b):
                    load(j, b).wait()
                    pltpu.sync_copy(bufs[b], acc.at[idx_v.at[j]], add=True)
            return carry

        lax.fori_loop(0, niter, step, 0)
        plsc.subcore_barrier()
        pltpu.sync_copy(acc.at[pl.ds(s * rps, rps)],
                        out_hbm.at[c, pl.ds(s * rps, rps)])

    f = pl.kernel(
        body,
        out_type=jax.ShapeDtypeStruct((NC, np_, d), jnp.float32),
        mesh=plsc.VectorSubcoreMesh(core_axis_name="c", subcore_axis_name="s"),
        scratch_types=(
            [pltpu.VMEM((niter, ch), jnp.int32)]
            + [pltpu.VMEM((ch, d), jnp.float32)] * NBUF_S
            + [pltpu.SemaphoreType.DMA] * NBUF_S
            + [pltpu.VMEM_SHARED((np_, d), jnp.float32)]
        ),
    )
    return f(vals, recv3, zeros)


def _sc_gather(tbl, snd3, e, ch):
    n, dt = tbl.shape
    niter = e // (NW * ch)

    def body(tbl_hbm, snd_hbm, out_hbm, idx_v, *rest):
        bufs = rest[:NBUF]
        gsems = rest[NBUF:2 * NBUF]
        wsems = rest[2 * NBUF:3 * NBUF]
        c = lax.axis_index("c")
        s = lax.axis_index("s")
        wid = c * NS + s
        base = wid * (niter * ch)
        pltpu.sync_copy(snd_hbm.at[wid], idx_v)

        def rd(j, b):
            return pltpu.make_async_copy(tbl_hbm.at[idx_v.at[j]],
                                         bufs[b], gsems[b])

        def wr(j, b):
            return pltpu.make_async_copy(
                bufs[b], out_hbm.at[pl.ds(base + j * ch, ch)], wsems[b])

        for b in range(NBUF - 1):
            rd(b, b).start()

        def step(j, carry):
            pre = j + NBUF - 1
            for b in range(NBUF):
                @pl.when(jnp.logical_and(pre < niter, pre % NBUF == b))
                def _(b=b):
                    @pl.when(pre >= NBUF)
                    def _():
                        wr(pre - NBUF, b).wait()
                    rd(pre, b).start()
            for b in range(NBUF):
                @pl.when(j % NBUF == b)
                def _(b=b):
                    rd(j, b).wait()
                    wr(j, b).start()
            return carry

        lax.fori_loop(0, niter, step, 0)
        for j in range(max(0, niter - NBUF), niter):
            wr(j, j % NBUF).wait()

    f = pl.kernel(
        body,
        out_type=jax.ShapeDtypeStruct((e, dt), jnp.float32),
        mesh=plsc.VectorSubcoreMesh(core_axis_name="c", subcore_axis_name="s"),
        scratch_types=(
            [pltpu.VMEM((niter, ch), jnp.int32)]
            + [pltpu.VMEM((ch, dt), jnp.float32)] * NBUF
            + [pltpu.SemaphoreType.DMA] * (2 * NBUF)
        ),
    )
    return f(tbl, snd3)


def _tc_tables(h_nodes, agg_e, w1ma, w1eb):
    n, d = h_nodes.shape
    bn = 1000

    def body(hn, pp, wa, wb, out):
        a = _mm(hn[...], wa[...])
        p = _mm(pp[0] + pp[1], wb[...])
        au = lax.bitcast_convert_type(a.astype(jnp.bfloat16),
                                      jnp.uint16).astype(jnp.uint32)
        pu = lax.bitcast_convert_type(p.astype(jnp.bfloat16),
                                      jnp.uint16).astype(jnp.uint32)
        out[...] = lax.bitcast_convert_type((au << 16) | pu, jnp.float32)

    row = pl.BlockSpec((bn, d), lambda i: (i, 0))
    psp = pl.BlockSpec((2, bn, d), lambda i: (0, i, 0))
    wsp = pl.BlockSpec((d, d), lambda i: (0, 0))
    return pl.pallas_call(
        body,
        grid=(n // bn,),
        in_specs=[row, psp, wsp, wsp],
        out_specs=row,
        out_shape=jax.ShapeDtypeStruct((n, d), jnp.float32),
    )(h_nodes, agg_e, w1ma, w1eb)


def _tc_edge_mlps(g, h_edges, wm, offb, eo_prev):
    e2, d = g.shape
    e = h_edges.shape[0]
    be = 2000

    def body(g_ref, he_ref, wcat, b1m, w2m, b2m, w3m, b3m, gm, bm,
             b1e, w2e, b2e, w3e, b3e, ge, be_, *rest):
        msg_ref, eo_ref = rest[-2], rest[-1]
        he = he_ref[...]
        hw = _mm(he, wcat[...])
        gw = lax.bitcast_convert_type(g_ref[...], jnp.uint32)
        ga = lax.bitcast_convert_type((gw >> 16).astype(jnp.uint16),
                                      jnp.bfloat16).astype(jnp.float32)
        gp = lax.bitcast_convert_type(gw.astype(jnp.uint16),
                                      jnp.bfloat16).astype(jnp.float32)
        x = jnp.maximum(ga + hw[:, :d] + b1m[...], 0.)
        x = jnp.maximum(_mm(x, w2m[...]) + b2m[...], 0.)
        x = _mm(x, w3m[...]) + b3m[...]
        msg_ref[...] = _ln(x, gm[...], bm[...])
        y = jnp.maximum(gp + hw[:, d:] + b1e[...], 0.)
        y = jnp.maximum(_mm(y, w2e[...]) + b2e[...], 0.)
        y = _mm(y, w3e[...]) + b3e[...]
        eo_ref[...] = he + _ln(y, ge[...], be_[...])

    row = pl.BlockSpec((be, d), lambda i: (i, 0))
    rowo = pl.BlockSpec((be, d), lambda i: (i + offb, 0))
    wsp2 = pl.BlockSpec((2 * d, d), lambda i: (0, 0))
    wsp = pl.BlockSpec((d, d), lambda i: (0, 0))
    vsp = pl.BlockSpec((1, d), lambda i: (0, 0))
    specs = [row, rowo, wsp2,
             vsp, wsp, vsp, wsp, vsp, vsp, vsp,
             vsp, wsp, vsp, wsp, vsp, vsp, vsp]
    args = [g, h_edges] + list(wm)
    kwargs = {}
    if eo_prev is not None:
        specs.append(pl.BlockSpec(memory_space=pl.ANY))
        args.append(eo_prev)
        kwargs["input_output_aliases"] = {17: 1}
    return pl.pallas_call(
        body,
        grid=(e2 // be,),
        in_specs=specs,
        out_specs=[row, rowo],
        out_shape=[jax.ShapeDtypeStruct((e2, d), jnp.float32),
                   jax.ShapeDtypeStruct((e, d), jnp.float32)],
        **kwargs,
    )(*args)


def _tc_node_mlp(h_nodes, am1, am2, wn):
    n, d = h_nodes.shape
    bn = 1000

    def body(hn_ref, q0_ref, q1_ref,
             w1na, w1nb, b1, w2, b2, w3, b3, gg, bb, out_ref):
        hn = hn_ref[...]
        q = (q0_ref[0] + q0_ref[1]) + (q1_ref[0] + q1_ref[1])
        x = jnp.maximum(_mm(hn, w1na[...]) + _mm(q, w1nb[...]) + b1[...], 0.)
        x = jnp.maximum(_mm(x, w2[...]) + b2[...], 0.)
        x = _mm(x, w3[...]) + b3[...]
        out_ref[...] = hn + _ln(x, gg[...], bb[...])

    row = pl.BlockSpec((bn, d), lambda i: (i, 0))
    psp = pl.BlockSpec((2, bn, d), lambda i: (0, i, 0))
    wsp = pl.BlockSpec((d, d), lambda i: (0, 0))
    vsp = pl.BlockSpec((1, d), lambda i: (0, 0))
    return pl.pallas_call(
        body,
        grid=(n // bn,),
        in_specs=[row, psp, psp,
                  wsp, wsp, vsp, wsp, vsp, wsp, vsp, vsp, vsp],
        out_specs=row,
        out_shape=jax.ShapeDtypeStruct((n, d), jnp.float32),
    )(h_nodes, am1, am2, *wn)


def kernel(h_nodes, h_edges, edge_index, params):
    n, d = h_nodes.shape
    e = h_edges.shape[0]
    e2 = e // 2
    ch_full, ch_half = 80, 40
    snd, rcv = edge_index[0], edge_index[1]
    rcv3 = rcv.reshape(NW, e // (NW * ch_full), ch_full)
    snd3a = snd[:e2].reshape(NW, e2 // (NW * ch_half), ch_half)
    snd3b = snd[e2:].reshape(NW, e2 // (NW * ch_half), ch_half)
    rcv3a = rcv[:e2].reshape(NW, e2 // (NW * ch_half), ch_half)
    rcv3b = rcv[e2:].reshape(NW, e2 // (NW * ch_half), ch_half)
    np_ = -(-n // (NS * 8)) * NS * 8
    zeros = jnp.zeros((np_ // NS, d), jnp.float32)

    pm, pn, pe = params["message"], params["node"], params["edge"]
    r2 = lambda v: v.reshape(1, d)

    agg_e = _sc_segment_sum(h_edges, rcv3, zeros, np_, ch_full)
    tbl = _tc_tables(h_nodes, agg_e, pm["W1"][:, :d], pe["W1"][:, d:])
    g1 = _sc_gather(tbl, snd3a, e2, ch_half)
    g2 = _sc_gather(tbl, snd3b, e2, ch_half)
    wcat = jnp.concatenate([pm["W1"][:, d:],
                            pe["W1"][:, :d] - pe["W1"][:, d:]], axis=0)
    wm = (wcat, r2(pm["b1"]), pm["W2"], r2(pm["b2"]),
          pm["W3"], r2(pm["b3"]), r2(pm["ln_g"]), r2(pm["ln_b"]),
          r2(pe["b1"]), pe["W2"], r2(pe["b2"]), pe["W3"], r2(pe["b3"]),
          r2(pe["ln_g"]), r2(pe["ln_b"]))
    msgs1, eo1 = _tc_edge_mlps(g1, h_edges, wm, 0, None)
    msgs2, h_edges_out = _tc_edge_mlps(g2, h_edges, wm, e2 // 2000, eo1)
    am1 = _sc_segment_sum(msgs1, rcv3a, zeros, np_, ch_half)
    am2 = _sc_segment_sum(msgs2, rcv3b, zeros, np_, ch_half)
    wn = (pn["W1"][:, :d], pn["W1"][:, d:], r2(pn["b1"]),
          pn["W2"], r2(pn["b2"]), pn["W3"], r2(pn["b3"]),
          r2(pn["ln_g"]), r2(pn["ln_b"]))
    h_nodes_out = _tc_node_mlp(h_nodes, am1, am2, wn)
    return (h_nodes_out, h_edges_out)
